# Initial kernel scaffold; baseline (speedup 1.0000x reference)
import jax
import jax.numpy as jnp
from jax.experimental import pallas as pl


def kernel(pred, label, h_state, h_faces):
    raise NotImplementedError("write your pallas kernel here")


# speedup vs baseline: 10.6049x; 10.6049x over previous
import functools

import jax
import jax.numpy as jnp
from jax import lax
from jax.experimental import pallas as pl
from jax.experimental.pallas import tpu as pltpu
from jax.experimental.pallas import tpu_sc as plsc

EPS = 1e-07
THRESH = 0.001
RADIUS = 0.05
BIGF = 1e9

BATCH = 2
NQ = 2048
NV = 6890
NVP = 7168
NF = 13776
NC, NSUB, LANES = 2, 16, 16
NW = NC * NSUB
FPT = 512
NFP = NW * FPT
ACC = 4 * NVP
RED = ACC // NSUB


def _rsqrt_nr(x):
    i = plsc.bitcast(x, jnp.int32)
    i = jnp.int32(0x5F3759DF) - lax.shift_right_logical(i, 1)
    y = plsc.bitcast(i, jnp.float32)
    for _ in range(3):
        y = y * (1.5 - 0.5 * x * y * y)
    return y


def _sqrt_sc(x):
    xc = jnp.maximum(x, 1e-30)
    return xc * _rsqrt_nr(xc)


def _vn_body(verts_hbm, faces_hbm, out_hbm, vloc, floc, sums, tmp, accv, shared):
    cid = lax.axis_index("c")
    sid = lax.axis_index("s")
    wid = sid * NC + cid
    ii = lax.iota(jnp.int32, 16)
    c0 = ii * 0
    zero = c0.astype(jnp.float32)
    one = zero + 1.0

    for b in range(BATCH):
        for c in range(3):
            pltpu.sync_copy(verts_hbm.at[b, 0, pl.ds(c * NVP, NVP)],
                            vloc.at[pl.ds(c * NVP, NVP)])
            pltpu.sync_copy(faces_hbm.at[b, 0, pl.ds(c * NFP + wid * FPT, FPT)],
                            floc.at[pl.ds(c * FPT, FPT)])

        def _zero(i, _):
            sums[pl.ds(i * 16, 16)] = zero
            return _
        lax.fori_loop(0, ACC // 16, _zero, 0)

        for j in range(FPT // 16):
            base = j * 16
            i0 = floc[pl.ds(base, 16)]
            i1 = floc[pl.ds(FPT + base, 16)]
            i2 = floc[pl.ds(2 * FPT + base, 16)]
            valid = (wid * FPT + base + ii) < NF
            v0x = plsc.load_gather(vloc, [i0])
            v0y = plsc.load_gather(vloc, [i0 + NVP])
            v0z = plsc.load_gather(vloc, [i0 + 2 * NVP])
            v1x = plsc.load_gather(vloc, [i1])
            v1y = plsc.load_gather(vloc, [i1 + NVP])
            v1z = plsc.load_gather(vloc, [i1 + 2 * NVP])
            v2x = plsc.load_gather(vloc, [i2])
            v2y = plsc.load_gather(vloc, [i2 + NVP])
            v2z = plsc.load_gather(vloc, [i2 + 2 * NVP])
            ax, ay, az = v1x - v0x, v1y - v0y, v1z - v0z
            bx, by, bz = v2x - v0x, v2y - v0y, v2z - v0z
            nx = ay * bz - az * by
            ny = az * bx - ax * bz
            nz = ax * by - ay * bx
            inv = 1.0 / (_sqrt_sc(nx * nx + ny * ny + nz * nz) + EPS)
            nx, ny, nz = nx * inv, ny * inv, nz * inv
            for idx in (i0, i1, i2):
                plsc.addupdate_scatter(sums, [idx], nx, mask=valid)
                plsc.addupdate_scatter(sums, [idx + NVP], ny, mask=valid)
                plsc.addupdate_scatter(sums, [idx + 2 * NVP], nz, mask=valid)
                plsc.addupdate_scatter(sums, [idx + 3 * NVP], one, mask=valid)

        pltpu.sync_copy(sums, shared.at[b, sid])

    plsc.subcore_barrier()

    for b in range(BATCH):
        pltpu.sync_copy(shared.at[b, 0, pl.ds(sid * RED, RED)], accv)
        for s in range(1, NSUB):
            pltpu.sync_copy(shared.at[b, s, pl.ds(sid * RED, RED)], tmp)

            def _acc(i, _):
                accv[pl.ds(i * 16, 16)] = accv[pl.ds(i * 16, 16)] + tmp[pl.ds(i * 16, 16)]
                return _
            lax.fori_loop(0, RED // 16, _acc, 0)
        pltpu.sync_copy(accv,
                        out_hbm.at[b, 0, pl.ds(cid * ACC + sid * RED, RED)])


@functools.partial(jax.jit)
def _vertex_normal_partials(verts_pad, faces_pad):
    mesh = plsc.VectorSubcoreMesh(
        core_axis_name="c", subcore_axis_name="s",
        num_cores=NC, num_subcores=NSUB)
    return pl.kernel(
        _vn_body,
        compiler_params=pltpu.CompilerParams(needs_layout_passes=False),
        out_type=jax.ShapeDtypeStruct((BATCH, 1, NC * ACC), jnp.float32),
        mesh=mesh,
        scratch_types=[
            pltpu.VMEM((3 * NVP,), jnp.float32),
            pltpu.VMEM((3 * FPT,), jnp.int32),
            pltpu.VMEM((ACC,), jnp.float32),
            pltpu.VMEM((RED,), jnp.float32),
            pltpu.VMEM((RED,), jnp.float32),
            pltpu.VMEM_SHARED((BATCH, NSUB, ACC), jnp.float32),
        ],
    )(verts_pad, faces_pad)


def _bq_body(pred_ref, verts_ref, out_ref):
    q = pred_ref[0]
    v = verts_ref[0]
    qx, qy, qz = q[:, 0:1], q[:, 1:2], q[:, 2:3]
    vx, vy, vz = v[0:1, :], v[1:2, :], v[2:3, :]
    q2 = qx * qx + qy * qy + qz * qz
    v2 = vx * vx + vy * vy + vz * vz
    qxb, qyb, qzb = (t.astype(jnp.bfloat16).astype(jnp.float32)
                     for t in (qx, qy, qz))
    vxb, vyb, vzb = (t.astype(jnp.bfloat16).astype(jnp.float32)
                     for t in (vx, vy, vz))
    d2 = (q2 + v2) - 2.0 * (qxb * vxb + qyb * vyb + qzb * vzb)
    niota = lax.broadcasted_iota(jnp.int32, (128, NVP), 1).astype(jnp.float32)
    keys = jnp.where(d2 < RADIUS * RADIUS, niota, BIGF)
    ms = []
    for j in range(4):
        m = jnp.min(keys, axis=1, keepdims=True)
        ms.append(m)
        if j < 3:
            keys = jnp.where(keys == m, BIGF, keys)
    fb = jnp.where(ms[0] < BIGF, ms[0], 0.0)
    cols = [jnp.where(m < BIGF, m, fb) for m in ms]
    out_ref[0] = jnp.concatenate(cols, axis=1).astype(jnp.int32)


@functools.partial(jax.jit)
def _ball_query_tc(pred, verts_pad):
    return pl.pallas_call(
        _bq_body,
        grid=(BATCH, NQ // 128),
        in_specs=[
            pl.BlockSpec((1, 128, 3), lambda b, s: (b, s, 0)),
            pl.BlockSpec((1, 3, NVP), lambda b, s: (b, 0, 0)),
        ],
        out_specs=pl.BlockSpec((1, 128, 4), lambda b, s: (b, s, 0)),
        out_shape=jax.ShapeDtypeStruct((BATCH, NQ, 4), jnp.int32),
    )(pred, verts_pad)


def _cr_body(verts_hbm, predt_hbm, idxt_hbm, part_hbm,
             pp_hbm, am_hbm, vloc, p0, p1, ploc, iloc, obuf):
    cid = lax.axis_index("c")
    sid = lax.axis_index("s")
    wid = sid * NC + cid
    b = wid // 16
    qoff = (wid % 16) * 128
    ii = lax.iota(jnp.int32, 16)
    c0 = ii * 0
    zero = c0.astype(jnp.float32)

    for c in range(3):
        pltpu.sync_copy(verts_hbm.at[b, 0, pl.ds(c * NVP, NVP)],
                        vloc.at[pl.ds(c * NVP, NVP)])
        pltpu.sync_copy(predt_hbm.at[b, 0, pl.ds(c * NQ + qoff, 128)],
                        ploc.at[pl.ds(c * 128, 128)])
    for k in range(4):
        pltpu.sync_copy(idxt_hbm.at[b, 0, pl.ds(k * NQ + qoff, 128)],
                        iloc.at[pl.ds(k * 128, 128)])
    pltpu.sync_copy(part_hbm.at[b, 0, pl.ds(0, ACC)], p0)
    pltpu.sync_copy(part_hbm.at[b, 0, pl.ds(ACC, ACC)], p1)

    acc_pp = zero
    acc_am = zero
    for j in range(8):
        base = j * 16
        qx = ploc[pl.ds(base, 16)]
        qy = ploc[pl.ds(128 + base, 16)]
        qz = ploc[pl.ds(256 + base, 16)]
        num = zero
        den = zero
        for k in range(4):
            idx = iloc[pl.ds(k * 128 + base, 16)]
            gx = plsc.load_gather(vloc, [idx])
            gy = plsc.load_gather(vloc, [idx + NVP])
            gz = plsc.load_gather(vloc, [idx + 2 * NVP])
            sx = plsc.load_gather(p0, [idx]) + plsc.load_gather(p1, [idx])
            sy = (plsc.load_gather(p0, [idx + NVP])
                  + plsc.load_gather(p1, [idx + NVP]))
            sz = (plsc.load_gather(p0, [idx + 2 * NVP])
                  + plsc.load_gather(p1, [idx + 2 * NVP]))
            cn = (plsc.load_gather(p0, [idx + 3 * NVP])
                  + plsc.load_gather(p1, [idx + 3 * NVP]))
            invc = 1.0 / (cn + EPS)
            nx, ny, nz = sx * invc, sy * invc, sz * invc
            invn = 1.0 / (_sqrt_sc(nx * nx + ny * ny + nz * nz) + EPS)
            nx, ny, nz = nx * invn, ny * invn, nz * invn
            dot = (qx - gx) * nx + (qy - gy) * ny + (qz - gz) * nz
            vmask = jnp.where(dot >= -RADIUS, 1.0, 0.0)
            vd = (dot - THRESH) * vmask
            neg = jnp.where(vd < 0.0, 1.0, 0.0)
            num = num + vd * neg
            den = den + neg
        t = num / (den + EPS)
        pp = t * t
        acc_pp = acc_pp + pp
        acc_am = acc_am + jnp.where(pp > 0.0, 1.0, 0.0)

    for j in range(16):
        obuf[pl.ds(j * 16, 16)] = zero
    obuf[pl.ds(0, 16)] = acc_pp
    obuf[pl.ds(128, 16)] = acc_am
    pltpu.sync_copy(obuf.at[pl.ds(0, 128)], pp_hbm.at[pl.ds(wid * 128, 128)])
    pltpu.sync_copy(obuf.at[pl.ds(128, 128)], am_hbm.at[pl.ds(wid * 128, 128)])


@functools.partial(jax.jit)
def _collision_reduce(verts_pad, predt, idxt, partials):
    mesh = plsc.VectorSubcoreMesh(
        core_axis_name="c", subcore_axis_name="s",
        num_cores=NC, num_subcores=NSUB)
    return pl.kernel(
        _cr_body,
        compiler_params=pltpu.CompilerParams(needs_layout_passes=False),
        out_type=(jax.ShapeDtypeStruct((NW * 128,), jnp.float32),
                  jax.ShapeDtypeStruct((NW * 128,), jnp.float32)),
        mesh=mesh,
        scratch_types=[
            pltpu.VMEM((3 * NVP,), jnp.float32),
            pltpu.VMEM((ACC,), jnp.float32),
            pltpu.VMEM((ACC,), jnp.float32),
            pltpu.VMEM((3 * 128,), jnp.float32),
            pltpu.VMEM((4 * 128,), jnp.int32),
            pltpu.VMEM((2 * 128,), jnp.float32),
        ],
    )(verts_pad, predt, idxt, partials)


def kernel(pred, label, h_state, h_faces):
    del label
    verts = h_state[:, :3, :]
    verts_pad = jnp.pad(verts, ((0, 0), (0, 0), (0, NVP - NV)),
                        constant_values=1000.0)
    faces = h_faces.astype(jnp.int32)
    faces_pad = jnp.pad(faces, ((0, 0), (0, 0), (0, NFP - NF)))
    predt = jnp.transpose(pred, (0, 2, 1))

    verts_sc = jnp.reshape(verts_pad, (BATCH, 1, 3 * NVP))
    faces_sc = jnp.reshape(faces_pad, (BATCH, 1, 3 * NFP))
    predt_sc = jnp.reshape(predt, (BATCH, 1, 3 * NQ))

    partials = _vertex_normal_partials(verts_sc, faces_sc)
    idx = _ball_query_tc(pred, verts_pad)
    idxt_sc = jnp.reshape(jnp.transpose(idx, (0, 2, 1)), (BATCH, 1, 4 * NQ))
    pp, am = _collision_reduce(verts_sc, predt_sc, idxt_sc, partials)
    return jnp.sum(pp) / (jnp.sum(am) + EPS)


# speedup vs baseline: 12.4082x; 1.1700x over previous
import functools

import jax
import jax.numpy as jnp
from jax import lax
from jax.experimental import pallas as pl
from jax.experimental.pallas import tpu as pltpu
from jax.experimental.pallas import tpu_sc as plsc

EPS = 1e-07
THRESH = 0.001
RADIUS = 0.05
BIGF = 1e9

BATCH = 2
NQ = 2048
NV = 6890
NVP = 7168
NF = 13776
NC, NSUB, LANES = 2, 16, 16
NW = NC * NSUB
FPT = 512
NFP = NW * FPT
ACC = 4 * NVP
RED = ACC // NSUB


def _rsqrt_nr(x):
    i = plsc.bitcast(x, jnp.int32)
    i = jnp.int32(0x5F3759DF) - lax.shift_right_logical(i, 1)
    y = plsc.bitcast(i, jnp.float32)
    for _ in range(3):
        y = y * (1.5 - 0.5 * x * y * y)
    return y


def _sqrt_sc(x):
    xc = jnp.maximum(x, 1e-30)
    return xc * _rsqrt_nr(xc)


def _vn_body(verts_hbm, faces_hbm, out_hbm, vloc, floc, sums, accv, pbuf,
             shared, sem):
    cid = lax.axis_index("c")
    sid = lax.axis_index("s")
    wid = sid * NC + cid
    ii = lax.iota(jnp.int32, 16)
    zero = (ii * 0).astype(jnp.float32)
    one = zero + 1.0

    def _zero_sums():
        def _zr(i, carry):
            for c in range(16):
                sums[pl.ds(i * 256 + c * 16, 16)] = zero
            return carry
        lax.fori_loop(0, ACC // 256, _zr, 0)

    for b in range(BATCH):
        for c in range(3):
            pltpu.sync_copy(verts_hbm.at[b, 0, pl.ds(c * NVP, NVP)],
                            vloc.at[pl.ds(c * NVP, NVP)])
            pltpu.sync_copy(faces_hbm.at[b, 0, pl.ds(c * NFP + wid * FPT, FPT)],
                            floc.at[pl.ds(c * FPT, FPT)])
        _zero_sums()

        def _chunk(j, carry):
            base = j * 16
            i0 = floc[pl.ds(base, 16)]
            i1 = floc[pl.ds(FPT + base, 16)]
            i2 = floc[pl.ds(2 * FPT + base, 16)]
            valid = (wid * FPT + base + ii) < NF
            v0x = plsc.load_gather(vloc, [i0])
            v0y = plsc.load_gather(vloc, [i0 + NVP])
            v0z = plsc.load_gather(vloc, [i0 + 2 * NVP])
            v1x = plsc.load_gather(vloc, [i1])
            v1y = plsc.load_gather(vloc, [i1 + NVP])
            v1z = plsc.load_gather(vloc, [i1 + 2 * NVP])
            v2x = plsc.load_gather(vloc, [i2])
            v2y = plsc.load_gather(vloc, [i2 + NVP])
            v2z = plsc.load_gather(vloc, [i2 + 2 * NVP])
            ax, ay, az = v1x - v0x, v1y - v0y, v1z - v0z
            bx, by, bz = v2x - v0x, v2y - v0y, v2z - v0z
            nx = ay * bz - az * by
            ny = az * bx - ax * bz
            nz = ax * by - ay * bx
            inv = 1.0 / (_sqrt_sc(nx * nx + ny * ny + nz * nz) + EPS)
            nx, ny, nz = nx * inv, ny * inv, nz * inv
            for idx in (i0, i1, i2):
                plsc.addupdate_scatter(sums, [idx], nx, mask=valid)
                plsc.addupdate_scatter(sums, [idx + NVP], ny, mask=valid)
                plsc.addupdate_scatter(sums, [idx + 2 * NVP], nz, mask=valid)
                plsc.addupdate_scatter(sums, [idx + 3 * NVP], one, mask=valid)
            return carry
        lax.fori_loop(0, FPT // 16, _chunk, 0)

        pltpu.sync_copy(sums, shared.at[b, sid])

    plsc.subcore_barrier()

    for b in range(BATCH):
        pltpu.sync_copy(shared.at[b, 0, pl.ds(sid * RED, RED)], accv)
        d = pltpu.async_copy(shared.at[b, 1, pl.ds(sid * RED, RED)],
                             pbuf.at[pl.ds(0, RED)], sem)
        for s in range(1, NSUB):
            d.wait()
            if s + 1 < NSUB:
                nbase = (s % 2) * RED
                d = pltpu.async_copy(shared.at[b, s + 1, pl.ds(sid * RED, RED)],
                                     pbuf.at[pl.ds(nbase, RED)], sem)
            cbase = ((s - 1) % 2) * RED

            def _acc(i, carry):
                for c in range(8):
                    off = pl.ds(i * 128 + c * 16, 16)
                    coff = pl.ds(cbase + i * 128 + c * 16, 16)
                    accv[off] = accv[off] + pbuf[coff]
                return carry
            lax.fori_loop(0, RED // 128, _acc, 0)
        pltpu.sync_copy(accv,
                        out_hbm.at[b, 0, pl.ds(cid * ACC + sid * RED, RED)])


@functools.partial(jax.jit)
def _vertex_normal_partials(verts_pad, faces_pad):
    mesh = plsc.VectorSubcoreMesh(
        core_axis_name="c", subcore_axis_name="s",
        num_cores=NC, num_subcores=NSUB)
    return pl.kernel(
        _vn_body,
        compiler_params=pltpu.CompilerParams(needs_layout_passes=False),
        out_type=jax.ShapeDtypeStruct((BATCH, 1, NC * ACC), jnp.float32),
        mesh=mesh,
        scratch_types=[
            pltpu.VMEM((3 * NVP,), jnp.float32),
            pltpu.VMEM((3 * FPT,), jnp.int32),
            pltpu.VMEM((ACC,), jnp.float32),
            pltpu.VMEM((RED,), jnp.float32),
            pltpu.VMEM((2 * RED,), jnp.float32),
            pltpu.VMEM_SHARED((BATCH, NSUB, ACC), jnp.float32),
            pltpu.SemaphoreType.DMA,
        ],
    )(verts_pad, faces_pad)


def _bq_body(pred_ref, verts_ref, out_ref):
    q = pred_ref[0]
    v = verts_ref[0]
    qx, qy, qz = q[:, 0:1], q[:, 1:2], q[:, 2:3]
    vx, vy, vz = v[0:1, :], v[1:2, :], v[2:3, :]
    q2 = qx * qx + qy * qy + qz * qz
    v2 = vx * vx + vy * vy + vz * vz
    cross = jnp.dot(q.astype(jnp.bfloat16), v.astype(jnp.bfloat16),
                    preferred_element_type=jnp.float32)
    d2 = (q2 + v2) - 2.0 * cross
    niota = lax.broadcasted_iota(jnp.int32, (128, NVP), 1).astype(jnp.float32)
    keys = jnp.where(d2 < RADIUS * RADIUS, niota, BIGF)
    ms = [jnp.min(keys, axis=1, keepdims=True)]
    for j in range(3):
        ms.append(jnp.min(jnp.where(keys <= ms[-1], BIGF, keys),
                          axis=1, keepdims=True))
    fb = jnp.where(ms[0] < BIGF, ms[0], 0.0)
    cols = [jnp.where(m < BIGF, m, fb) for m in ms]
    out_ref[0] = jnp.concatenate(cols, axis=1).astype(jnp.int32)


@functools.partial(jax.jit)
def _ball_query_tc(pred, verts_pad):
    return pl.pallas_call(
        _bq_body,
        grid=(BATCH, NQ // 128),
        in_specs=[
            pl.BlockSpec((1, 128, 3), lambda b, s: (b, s, 0)),
            pl.BlockSpec((1, 3, NVP), lambda b, s: (b, 0, 0)),
        ],
        out_specs=pl.BlockSpec((1, 128, 4), lambda b, s: (b, s, 0)),
        out_shape=jax.ShapeDtypeStruct((BATCH, NQ, 4), jnp.int32),
    )(pred, verts_pad)


def _cr_body(verts_hbm, predt_hbm, idxt_hbm, part_hbm,
             pp_hbm, am_hbm, vloc, p0, p1, ploc, iloc, obuf):
    cid = lax.axis_index("c")
    sid = lax.axis_index("s")
    wid = sid * NC + cid
    b = wid // 16
    qoff = (wid % 16) * 128
    ii = lax.iota(jnp.int32, 16)
    c0 = ii * 0
    zero = c0.astype(jnp.float32)

    for c in range(3):
        pltpu.sync_copy(verts_hbm.at[b, 0, pl.ds(c * NVP, NVP)],
                        vloc.at[pl.ds(c * NVP, NVP)])
        pltpu.sync_copy(predt_hbm.at[b, 0, pl.ds(c * NQ + qoff, 128)],
                        ploc.at[pl.ds(c * 128, 128)])
    for k in range(4):
        pltpu.sync_copy(idxt_hbm.at[b, 0, pl.ds(k * NQ + qoff, 128)],
                        iloc.at[pl.ds(k * 128, 128)])
    pltpu.sync_copy(part_hbm.at[b, 0, pl.ds(0, ACC)], p0)
    pltpu.sync_copy(part_hbm.at[b, 0, pl.ds(ACC, ACC)], p1)

    acc_pp = zero
    acc_am = zero
    for j in range(8):
        base = j * 16
        qx = ploc[pl.ds(base, 16)]
        qy = ploc[pl.ds(128 + base, 16)]
        qz = ploc[pl.ds(256 + base, 16)]
        num = zero
        den = zero
        for k in range(4):
            idx = iloc[pl.ds(k * 128 + base, 16)]
            gx = plsc.load_gather(vloc, [idx])
            gy = plsc.load_gather(vloc, [idx + NVP])
            gz = plsc.load_gather(vloc, [idx + 2 * NVP])
            sx = plsc.load_gather(p0, [idx]) + plsc.load_gather(p1, [idx])
            sy = (plsc.load_gather(p0, [idx + NVP])
                  + plsc.load_gather(p1, [idx + NVP]))
            sz = (plsc.load_gather(p0, [idx + 2 * NVP])
                  + plsc.load_gather(p1, [idx + 2 * NVP]))
            cn = (plsc.load_gather(p0, [idx + 3 * NVP])
                  + plsc.load_gather(p1, [idx + 3 * NVP]))
            invc = 1.0 / (cn + EPS)
            nx, ny, nz = sx * invc, sy * invc, sz * invc
            invn = 1.0 / (_sqrt_sc(nx * nx + ny * ny + nz * nz) + EPS)
            nx, ny, nz = nx * invn, ny * invn, nz * invn
            dot = (qx - gx) * nx + (qy - gy) * ny + (qz - gz) * nz
            vmask = jnp.where(dot >= -RADIUS, 1.0, 0.0)
            vd = (dot - THRESH) * vmask
            neg = jnp.where(vd < 0.0, 1.0, 0.0)
            num = num + vd * neg
            den = den + neg
        t = num / (den + EPS)
        pp = t * t
        acc_pp = acc_pp + pp
        acc_am = acc_am + jnp.where(pp > 0.0, 1.0, 0.0)

    for j in range(16):
        obuf[pl.ds(j * 16, 16)] = zero
    obuf[pl.ds(0, 16)] = acc_pp
    obuf[pl.ds(128, 16)] = acc_am
    pltpu.sync_copy(obuf.at[pl.ds(0, 128)], pp_hbm.at[pl.ds(wid * 128, 128)])
    pltpu.sync_copy(obuf.at[pl.ds(128, 128)], am_hbm.at[pl.ds(wid * 128, 128)])


@functools.partial(jax.jit)
def _collision_reduce(verts_pad, predt, idxt, partials):
    mesh = plsc.VectorSubcoreMesh(
        core_axis_name="c", subcore_axis_name="s",
        num_cores=NC, num_subcores=NSUB)
    return pl.kernel(
        _cr_body,
        compiler_params=pltpu.CompilerParams(needs_layout_passes=False),
        out_type=(jax.ShapeDtypeStruct((NW * 128,), jnp.float32),
                  jax.ShapeDtypeStruct((NW * 128,), jnp.float32)),
        mesh=mesh,
        scratch_types=[
            pltpu.VMEM((3 * NVP,), jnp.float32),
            pltpu.VMEM((ACC,), jnp.float32),
            pltpu.VMEM((ACC,), jnp.float32),
            pltpu.VMEM((3 * 128,), jnp.float32),
            pltpu.VMEM((4 * 128,), jnp.int32),
            pltpu.VMEM((2 * 128,), jnp.float32),
        ],
    )(verts_pad, predt, idxt, partials)


def kernel(pred, label, h_state, h_faces):
    del label
    verts = h_state[:, :3, :]
    verts_pad = jnp.pad(verts, ((0, 0), (0, 0), (0, NVP - NV)),
                        constant_values=1000.0)
    faces = h_faces.astype(jnp.int32)
    faces_pad = jnp.pad(faces, ((0, 0), (0, 0), (0, NFP - NF)))
    predt = jnp.transpose(pred, (0, 2, 1))

    verts_sc = jnp.reshape(verts_pad, (BATCH, 1, 3 * NVP))
    faces_sc = jnp.reshape(faces_pad, (BATCH, 1, 3 * NFP))
    predt_sc = jnp.reshape(predt, (BATCH, 1, 3 * NQ))

    partials = _vertex_normal_partials(verts_sc, faces_sc)
    idx = _ball_query_tc(pred, verts_pad)
    idxt_sc = jnp.reshape(jnp.transpose(idx, (0, 2, 1)), (BATCH, 1, 4 * NQ))
    pp, am = _collision_reduce(verts_sc, predt_sc, idxt_sc, partials)
    return jnp.sum(pp) / (jnp.sum(am) + EPS)
